# baseline (speedup 1.0000x reference)
import functools

import jax
import jax.numpy as jnp
from jax import lax
from jax.experimental import pallas as pl
from jax.experimental.pallas import tpu as pltpu
from jax.experimental.pallas import tpu_sc as plsc

BATCH = 4096
SEQ = 200
DIM = 64
VOCAB = 1000000
B_TOTAL = BATCH * SEQ
NUM_WORKERS = 32
LANES = 16

VCOLS = VOCAB // 128
VREM = VOCAB - VCOLS * 128
A_STEPS = VCOLS // NUM_WORKERS + 1
A_PAIRS = (A_STEPS + 1) // 2

_MESH = plsc.VectorSubcoreMesh(core_axis_name="c", subcore_axis_name="s")


@functools.partial(
    pl.kernel,
    mesh=_MESH,
    out_type=jax.ShapeDtypeStruct((VOCAB * DIM,), jnp.float32),
    scratch_types=[
        pltpu.VMEM((DIM, 128), jnp.float32),
        pltpu.VMEM((DIM, 128), jnp.float32),
        pltpu.VMEM((128 * DIM,), jnp.float32),
        pltpu.VMEM((128 * DIM,), jnp.float32),
        pltpu.SemaphoreType.DMA,
        pltpu.SemaphoreType.DMA,
        pltpu.SemaphoreType.DMA,
        pltpu.SemaphoreType.DMA,
    ],
    compiler_params=pltpu.CompilerParams(use_tc_tiling_on_sc=True,
                                        needs_layout_passes=False),
)
def _transpose_kernel(wt_hbm, wrem_hbm, tbl_hbm, in0, in1, out0, out1,
                      wsem0, wsem1, rsem0, rsem1):
    wid = lax.axis_index("s") * 2 + lax.axis_index("c")

    base_idx = [(lax.iota(jnp.int32, LANES) + 16 * k) * DIM for k in range(8)]

    def transpose_block(blk_in, blk_out, ncols):
        @plsc.parallel_loop(0, 16, unroll=4)
        def body_r(r):
            for m in range(4):
                dst = blk_out.at[pl.ds(16 * m, (128 - 1) * DIM + 16)]
                for k in range(8):
                    if k * LANES < ncols:
                        v = blk_in[16 * m + r, pl.ds(16 * k, LANES)]
                        plsc.store_scatter(dst, [base_idx[k] + r], v)

    def writeback(c, blk_out, sem):
        return pltpu.make_async_copy(
            blk_out, tbl_hbm.at[pl.ds(c * (128 * DIM), 128 * DIM)], sem)

    def read(c, blk_in, sem):
        return pltpu.make_async_copy(
            wt_hbm.at[:, pl.ds(c * 128, 128)], blk_in, sem)

    read(wid, in0, rsem0).start()

    def do_col(c, blk_in, blk_out, rsem, wsem, first):
        @pl.when(c < VCOLS)
        def _():
            read(c, blk_in, rsem).wait()

            @pl.when(jnp.logical_not(first))
            def _():
                pltpu.make_async_copy(blk_out, tbl_hbm.at[pl.ds(0, 128 * DIM)],
                                      wsem).wait()
            transpose_block(blk_in, blk_out, 128)
            writeback(c, blk_out, wsem).start()

    def body(j, carry):
        c0 = (2 * j) * NUM_WORKERS + wid
        c1 = c0 + NUM_WORKERS
        c2 = c1 + NUM_WORKERS

        @pl.when(c1 < VCOLS)
        def _():
            read(c1, in1, rsem1).start()
        do_col(c0, in0, out0, rsem0, wsem0, j == 0)

        @pl.when(c2 < VCOLS)
        def _():
            read(c2, in0, rsem0).start()
        do_col(c1, in1, out1, rsem1, wsem1, j == 0)
        return carry

    lax.fori_loop(0, A_PAIRS, body, 0)

    pltpu.make_async_copy(out0, tbl_hbm.at[pl.ds(0, 128 * DIM)], wsem0).wait()
    pltpu.make_async_copy(out1, tbl_hbm.at[pl.ds(0, 128 * DIM)], wsem1).wait()

    @pl.when(wid == NUM_WORKERS - 1)
    def _():
        pltpu.sync_copy(wrem_hbm, in0)

        def body_l(l, carry):
            for k in range(4):
                out0[pl.ds(l * DIM + 16 * k, LANES)] = \
                    in0[l, pl.ds(16 * k, LANES)]
            return carry
        lax.fori_loop(0, VREM, body_l, 0)
        pltpu.sync_copy(out0.at[pl.ds(0, VREM * DIM)],
                        tbl_hbm.at[pl.ds(VCOLS * 128 * DIM, VREM * DIM)])


B_PER_W = B_TOTAL // NUM_WORKERS
CHUNK = 512
N_CHUNKS = B_PER_W // CHUNK
N_PAIRS = N_CHUNKS // 2
PDIM = 128


@functools.partial(
    pl.kernel,
    mesh=_MESH,
    out_type=jax.ShapeDtypeStruct((B_TOTAL, PDIM), jnp.float32),
    scratch_types=[
        pltpu.VMEM((N_CHUNKS, CHUNK), jnp.int32),
        pltpu.VMEM((CHUNK, DIM), jnp.float32),
        pltpu.VMEM((CHUNK, DIM), jnp.float32),
        pltpu.SemaphoreType.DMA,
        pltpu.SemaphoreType.DMA,
        pltpu.SemaphoreType.DMA,
        pltpu.SemaphoreType.DMA,
    ],
    compiler_params=pltpu.CompilerParams(use_tc_tiling_on_sc=False),
)
def _gather_kernel(idx_hbm, table_hbm, out_hbm, idx_v, rows0, rows1,
                   gsem0, gsem1, wsem0, wsem1):
    wid = lax.axis_index("s") * 2 + lax.axis_index("c")
    base = wid * B_PER_W

    pltpu.sync_copy(idx_hbm.at[wid], idx_v)

    def gather(c, rows, sem):
        return pltpu.make_async_copy(table_hbm.at[idx_v.at[c]], rows, sem)

    def writeback(c, rows, sem):
        return pltpu.make_async_copy(
            rows,
            out_hbm.at[pl.ds(base + c * CHUNK, CHUNK), pl.ds(0, DIM)],
            sem)

    gather(0, rows0, gsem0).start()

    def body(k, carry):
        c0 = 2 * k

        @pl.when(k > 0)
        def _():
            writeback(c0 - 1, rows1, wsem1).wait()

        gather(c0 + 1, rows1, gsem1).start()
        gather(c0, rows0, gsem0).wait()
        writeback(c0, rows0, wsem0).start()

        @pl.when(k < N_PAIRS - 1)
        def _():
            writeback(c0, rows0, wsem0).wait()
            gather(c0 + 2, rows0, gsem0).start()

        gather(c0 + 1, rows1, gsem1).wait()
        writeback(c0 + 1, rows1, wsem1).start()
        return carry

    lax.fori_loop(0, N_PAIRS, body, 0)

    writeback(N_CHUNKS - 2, rows0, wsem0).wait()
    writeback(N_CHUNKS - 1, rows1, wsem1).wait()


def kernel(input_, weight):
    wrem = jnp.pad(weight[VCOLS * 128:], ((0, 0), (0, 128 - DIM)))
    table1d = _transpose_kernel(weight.T, wrem)
    table = table1d.reshape(VOCAB, DIM)
    idx = input_.reshape(NUM_WORKERS, N_CHUNKS, CHUNK).astype(jnp.int32)
    out = _gather_kernel(idx, table)
    return out[:, :DIM].reshape(BATCH, SEQ, DIM)


# speedup vs baseline: 1.4185x; 1.4185x over previous
import functools

import jax
import jax.numpy as jnp
from jax import lax
from jax.experimental import pallas as pl
from jax.experimental.pallas import tpu as pltpu
from jax.experimental.pallas import tpu_sc as plsc

BATCH = 4096
SEQ = 200
DIM = 64
VOCAB = 1000000
B_TOTAL = BATCH * SEQ
NUM_WORKERS = 32
LANES = 16

_MESH = plsc.VectorSubcoreMesh(core_axis_name="c", subcore_axis_name="s")

B_PER_W = B_TOTAL // NUM_WORKERS
CHUNK = 512
N_CHUNKS = B_PER_W // CHUNK
N_PAIRS = N_CHUNKS // 2
PDIM = 128


@functools.partial(
    pl.kernel,
    mesh=_MESH,
    out_type=jax.ShapeDtypeStruct((B_TOTAL, PDIM), jnp.float32),
    scratch_types=[
        pltpu.VMEM((N_CHUNKS, CHUNK), jnp.int32),
        pltpu.VMEM((CHUNK, DIM), jnp.float32),
        pltpu.VMEM((CHUNK, DIM), jnp.float32),
        pltpu.SemaphoreType.DMA,
        pltpu.SemaphoreType.DMA,
        pltpu.SemaphoreType.DMA,
        pltpu.SemaphoreType.DMA,
    ],
    compiler_params=pltpu.CompilerParams(use_tc_tiling_on_sc=False),
)
def _gather_kernel(idx_hbm, table_hbm, out_hbm, idx_v, rows0, rows1,
                   gsem0, gsem1, wsem0, wsem1):
    wid = lax.axis_index("s") * 2 + lax.axis_index("c")
    base = wid * B_PER_W

    pltpu.sync_copy(idx_hbm.at[wid], idx_v)

    def gather(c, rows, sem):
        return pltpu.make_async_copy(table_hbm.at[idx_v.at[c]], rows, sem)

    def writeback(c, rows, sem):
        return pltpu.make_async_copy(
            rows,
            out_hbm.at[pl.ds(base + c * CHUNK, CHUNK), pl.ds(0, DIM)],
            sem)

    gather(0, rows0, gsem0).start()

    def body(k, carry):
        c0 = 2 * k

        @pl.when(k > 0)
        def _():
            writeback(c0 - 1, rows1, wsem1).wait()

        gather(c0 + 1, rows1, gsem1).start()
        gather(c0, rows0, gsem0).wait()
        writeback(c0, rows0, wsem0).start()

        @pl.when(k < N_PAIRS - 1)
        def _():
            writeback(c0, rows0, wsem0).wait()
            gather(c0 + 2, rows0, gsem0).start()

        gather(c0 + 1, rows1, gsem1).wait()
        writeback(c0 + 1, rows1, wsem1).start()
        return carry

    lax.fori_loop(0, N_PAIRS, body, 0)

    writeback(N_CHUNKS - 2, rows0, wsem0).wait()
    writeback(N_CHUNKS - 1, rows1, wsem1).wait()


def kernel(input_, weight):
    wpad = jnp.pad(weight, ((0, 0), (0, 128 - DIM)))
    table = wpad.reshape(2 * VOCAB, DIM)
    idx = (input_.reshape(NUM_WORKERS, N_CHUNKS, CHUNK) * 2).astype(jnp.int32)
    out = _gather_kernel(idx, table)
    return out[:, :DIM].reshape(BATCH, SEQ, DIM)
